# Initial kernel scaffold; baseline (speedup 1.0000x reference)
import jax
import jax.numpy as jnp
from jax.experimental import pallas as pl


def kernel(x, edge_index, batch, W1, b1, W2, b2, Wp1, bp1, Wp2, bp2):
    raise NotImplementedError("write your pallas kernel here")


# speedup vs baseline: 12.9545x; 12.9545x over previous
import functools

import jax
import jax.numpy as jnp
from jax import lax
from jax.experimental import pallas as pl
from jax.experimental.pallas import tpu as pltpu
from jax.experimental.pallas import tpu_sc as plsc

NS = 16
K = 128


def _mesh():
    return plsc.VectorSubcoreMesh(
        core_axis_name="c", subcore_axis_name="s", num_cores=1, num_subcores=NS
    )


CB = 40


@functools.lru_cache(maxsize=None)
def _make_aggregate_kernel(n, d, nbatch, gather):
    npad = n + 8
    rpt8 = (npad // NS) // 8 * 8
    zrem = npad - rpt8 * NS
    crem = n - rpt8 * NS

    scratch = [
        pltpu.VMEM((CB, K), jnp.int32),
        pltpu.VMEM((K, d), jnp.float32),
        pltpu.VMEM_SHARED((npad, d), jnp.float32),
    ]
    if gather:
        scratch.insert(1, pltpu.VMEM((CB, K), jnp.int32))
        scratch.insert(3, pltpu.VMEM((K, d), jnp.float32))
        scratch += [pltpu.SemaphoreType.DMA, pltpu.SemaphoreType.DMA]

    def zero_init(zeros_hbm, acc_sh, sid):
        row0 = sid * rpt8
        pltpu.sync_copy(zeros_hbm.at[pl.ds(row0, rpt8)], acc_sh.at[pl.ds(row0, rpt8)])

        @pl.when(sid == NS - 1)
        def _():
            pltpu.sync_copy(
                zeros_hbm.at[pl.ds(NS * rpt8, zrem)], acc_sh.at[pl.ds(NS * rpt8, zrem)]
            )

    def copy_out(acc_sh, out_hbm, sid):
        row0 = sid * rpt8
        pltpu.sync_copy(acc_sh.at[pl.ds(row0, rpt8)], out_hbm.at[pl.ds(row0, rpt8)])

        @pl.when(sid == NS - 1)
        def _():
            pltpu.sync_copy(
                acc_sh.at[pl.ds(NS * rpt8, crem)], out_hbm.at[pl.ds(NS * rpt8, crem)]
            )

    def body_gather(y_hbm, src_hbm, dst_hbm, zeros_hbm, out_hbm,
                    dst_v, src_v, m0, m1, acc_sh, sem0, sem1):
        sid = lax.axis_index("s")
        zero_init(zeros_hbm, acc_sh, sid)
        plsc.subcore_barrier()

        for b in range(nbatch):
            pltpu.sync_copy(src_hbm.at[sid, pl.ds(b * CB, CB)], src_v)
            pltpu.sync_copy(dst_hbm.at[sid, pl.ds(b * CB, CB)], dst_v)

            def loop(jj, carry):
                j0 = 2 * jj
                j1 = j0 + 1
                c0 = pltpu.async_copy(y_hbm.at[src_v.at[j0]], m0, sem0)
                c1 = pltpu.async_copy(y_hbm.at[src_v.at[j1]], m1, sem1)
                c0.wait()
                pltpu.sync_copy(m0, acc_sh.at[dst_v.at[j0]], add=True)
                c1.wait()
                pltpu.sync_copy(m1, acc_sh.at[dst_v.at[j1]], add=True)
                return carry

            lax.fori_loop(0, CB // 2, loop, 0)

        plsc.subcore_barrier()
        copy_out(acc_sh, out_hbm, sid)

    def body_ones(dst_hbm, zeros_hbm, out_hbm, dst_v, ones_v, acc_sh):
        sid = lax.axis_index("s")
        one = jnp.full((16,), 1.0, jnp.float32)

        def fill(i, carry):
            for q in range(d // 16):
                ones_v[i, pl.ds(q * 16, 16)] = one
            return carry

        lax.fori_loop(0, K, fill, 0)
        zero_init(zeros_hbm, acc_sh, sid)
        plsc.subcore_barrier()

        for b in range(nbatch):
            pltpu.sync_copy(dst_hbm.at[sid, pl.ds(b * CB, CB)], dst_v)

            def loop(j, carry):
                pltpu.sync_copy(ones_v, acc_sh.at[dst_v.at[j]], add=True)
                return carry

            lax.fori_loop(0, CB, loop, 0)

        plsc.subcore_barrier()
        copy_out(acc_sh, out_hbm, sid)

    return pl.kernel(
        body_gather if gather else body_ones,
        out_type=jax.ShapeDtypeStruct((n, d), jnp.float32),
        mesh=_mesh(),
        scratch_types=scratch,
    )


def _dinv_block(s_ones):
    return lax.rsqrt(s_ones[:, 0:1] + 1.0)


def _tc_prep(x, s_ones, block):
    n, d = x.shape

    def body(x_ref, so_ref, y_ref):
        y_ref[...] = x_ref[...] * _dinv_block(so_ref[...])

    spec = pl.BlockSpec((block, d), lambda i: (i, 0))
    return pl.pallas_call(
        body,
        grid=(n // block,),
        in_specs=[spec, spec],
        out_specs=spec,
        out_shape=jax.ShapeDtypeStruct((n, d), jnp.float32),
    )(x, s_ones)


def _tc_mid(s, y, s_ones, w, b, block):
    n, d = y.shape

    def body(s_ref, y_ref, so_ref, w_ref, b_ref, h_ref, yn_ref):
        dinv = _dinv_block(so_ref[...])
        agg = (s_ref[...] + y_ref[...]) * dinv
        h = jnp.dot(agg, w_ref[...], preferred_element_type=jnp.float32)
        h = h + b_ref[...]
        h_ref[...] = h
        yn_ref[...] = jnp.maximum(h, 0.0) * dinv

    spec = pl.BlockSpec((block, d), lambda i: (i, 0))
    return pl.pallas_call(
        body,
        grid=(n // block,),
        in_specs=[
            spec, spec, spec,
            pl.BlockSpec((d, d), lambda i: (0, 0)),
            pl.BlockSpec((1, d), lambda i: (0, 0)),
        ],
        out_specs=[spec, spec],
        out_shape=[
            jax.ShapeDtypeStruct((n, d), jnp.float32),
            jax.ShapeDtypeStruct((n, d), jnp.float32),
        ],
    )(s, y, s_ones, w, b.reshape(1, d))


def _tc_final(s, y, s_ones, w2, b2, wp1, bp1, wp2, bp2, block):
    n, d = y.shape

    def body(s_ref, y_ref, so_ref, w2_ref, b2_ref, wp1_ref, bp1_ref,
             wp2_ref, bp2_ref, h2_ref, out_ref):
        dinv = _dinv_block(so_ref[...])
        agg = (s_ref[...] + y_ref[...]) * dinv
        h2 = jnp.dot(agg, w2_ref[...], preferred_element_type=jnp.float32)
        h2 = h2 + b2_ref[...]
        h2_ref[...] = h2
        r = jnp.maximum(h2, 0.0)
        t = jnp.dot(r, wp1_ref[...], preferred_element_type=jnp.float32)
        t = t + bp1_ref[...]
        p = jnp.dot(t, wp2_ref[...], preferred_element_type=jnp.float32)
        p = p + bp2_ref[...]
        m = jnp.max(p, axis=1, keepdims=True)
        lse = jnp.log(jnp.sum(jnp.exp(p - m), axis=1, keepdims=True)) + m
        out_ref[...] = p - lse

    spec = pl.BlockSpec((block, d), lambda i: (i, 0))
    wspec = pl.BlockSpec((d, d), lambda i: (0, 0))
    bspec = pl.BlockSpec((1, d), lambda i: (0, 0))
    return pl.pallas_call(
        body,
        grid=(n // block,),
        in_specs=[spec, spec, spec, wspec, bspec, wspec, bspec, wspec, bspec],
        out_specs=[spec, spec],
        out_shape=[
            jax.ShapeDtypeStruct((n, d), jnp.float32),
            jax.ShapeDtypeStruct((n, d), jnp.float32),
        ],
    )(s, y, s_ones, w2, b2.reshape(1, d), wp1, bp1.reshape(1, d),
      wp2, bp2.reshape(1, d))


@jax.jit
def kernel(x, edge_index, batch, W1, b1, W2, b2, Wp1, bp1, Wp2, bp2):
    n, d = x.shape
    e = edge_index.shape[1]
    nbatch = -(-e // (NS * K * CB))
    chunks = nbatch * CB
    epad = chunks * NS * K - e
    pad_i = jnp.arange(epad, dtype=jnp.int32)
    src3 = jnp.concatenate([edge_index[0], pad_i % 64]).reshape(NS, chunks, K)
    dst3 = jnp.concatenate([edge_index[1], n + (pad_i % 8)]).reshape(NS, chunks, K)
    zeros = jnp.zeros((n + 8, d), jnp.float32)

    agg_ones = _make_aggregate_kernel(n, d, nbatch, False)
    agg = _make_aggregate_kernel(n, d, nbatch, True)

    block = 1000
    s_ones = agg_ones(dst3, zeros)
    y1 = _tc_prep(x, s_ones, block)
    s1 = agg(y1, src3, dst3, zeros)
    h1, y2 = _tc_mid(s1, y1, s_ones, W1, b1, block)
    s2 = agg(y2, src3, dst3, zeros)
    h2, out = _tc_final(s2, y2, s_ones, W2, b2, Wp1, bp1, Wp2, bp2, block)
    return (h1, h2, out)


# speedup vs baseline: 12.9700x; 1.0012x over previous
import functools

import jax
import jax.numpy as jnp
from jax import lax
from jax.experimental import pallas as pl
from jax.experimental.pallas import tpu as pltpu
from jax.experimental.pallas import tpu_sc as plsc

NS = 16
K = 128


def _mesh():
    return plsc.VectorSubcoreMesh(
        core_axis_name="c", subcore_axis_name="s", num_cores=1, num_subcores=NS
    )


CB = 40


@functools.lru_cache(maxsize=None)
def _make_aggregate_kernel(n, d, nbatch, gather):
    npad = n + 8
    rpt8 = (npad // NS) // 8 * 8
    zrem = npad - rpt8 * NS
    crem = n - rpt8 * NS

    scratch = [
        pltpu.VMEM((CB, K), jnp.int32),
        pltpu.VMEM((K, d), jnp.float32),
        pltpu.VMEM_SHARED((npad, d), jnp.float32),
    ]
    if gather:
        scratch.insert(1, pltpu.VMEM((CB, K), jnp.int32))
        scratch.insert(3, pltpu.VMEM((K, d), jnp.float32))
    scratch += [pltpu.SemaphoreType.DMA, pltpu.SemaphoreType.DMA]

    def zero_init(zeros_hbm, acc_sh, sid):
        row0 = sid * rpt8
        pltpu.sync_copy(zeros_hbm.at[pl.ds(row0, rpt8)], acc_sh.at[pl.ds(row0, rpt8)])

        @pl.when(sid == NS - 1)
        def _():
            pltpu.sync_copy(
                zeros_hbm.at[pl.ds(NS * rpt8, zrem)], acc_sh.at[pl.ds(NS * rpt8, zrem)]
            )

    def copy_out(acc_sh, out_hbm, sid):
        row0 = sid * rpt8
        pltpu.sync_copy(acc_sh.at[pl.ds(row0, rpt8)], out_hbm.at[pl.ds(row0, rpt8)])

        @pl.when(sid == NS - 1)
        def _():
            pltpu.sync_copy(
                acc_sh.at[pl.ds(NS * rpt8, crem)], out_hbm.at[pl.ds(NS * rpt8, crem)]
            )

    def body_gather(y_hbm, src_hbm, dst_hbm, zeros_hbm, out_hbm,
                    dst_v, src_v, m0, m1, acc_sh, sem0, sem1):
        sid = lax.axis_index("s")
        zero_init(zeros_hbm, acc_sh, sid)
        plsc.subcore_barrier()

        for b in range(nbatch):
            pltpu.sync_copy(src_hbm.at[sid, pl.ds(b * CB, CB)], src_v)
            pltpu.sync_copy(dst_hbm.at[sid, pl.ds(b * CB, CB)], dst_v)

            def loop(jj, carry):
                j0 = 2 * jj
                j1 = j0 + 1
                c0 = pltpu.async_copy(y_hbm.at[src_v.at[j0]], m0, sem0)
                c1 = pltpu.async_copy(y_hbm.at[src_v.at[j1]], m1, sem1)
                c0.wait()
                pltpu.sync_copy(m0, acc_sh.at[dst_v.at[j0]], add=True)
                c1.wait()
                pltpu.sync_copy(m1, acc_sh.at[dst_v.at[j1]], add=True)
                return carry

            lax.fori_loop(0, CB // 2, loop, 0)

        plsc.subcore_barrier()
        copy_out(acc_sh, out_hbm, sid)

    def body_ones(dst_hbm, zeros_hbm, out_hbm, dst_v, ones_v, acc_sh, sem0, sem1):
        sid = lax.axis_index("s")
        one = jnp.full((16,), 1.0, jnp.float32)

        def fill(i, carry):
            for q in range(d // 16):
                ones_v[i, pl.ds(q * 16, 16)] = one
            return carry

        lax.fori_loop(0, K, fill, 0)
        zero_init(zeros_hbm, acc_sh, sid)
        plsc.subcore_barrier()

        for b in range(nbatch):
            pltpu.sync_copy(dst_hbm.at[sid, pl.ds(b * CB, CB)], dst_v)

            def loop(jj, carry):
                j0 = 2 * jj
                c0 = pltpu.async_copy(ones_v, acc_sh.at[dst_v.at[j0]], sem0, add=True)
                c1 = pltpu.async_copy(ones_v, acc_sh.at[dst_v.at[j0 + 1]], sem1, add=True)
                c0.wait()
                c1.wait()
                return carry

            lax.fori_loop(0, CB // 2, loop, 0)

        plsc.subcore_barrier()
        copy_out(acc_sh, out_hbm, sid)

    return pl.kernel(
        body_gather if gather else body_ones,
        out_type=jax.ShapeDtypeStruct((n, d), jnp.float32),
        mesh=_mesh(),
        scratch_types=scratch,
    )


def _dinv_block(s_ones):
    return lax.rsqrt(s_ones[:, 0:1] + 1.0)


def _tc_prep(x, s_ones, block):
    n, d = x.shape

    def body(x_ref, so_ref, y_ref):
        y_ref[...] = x_ref[...] * _dinv_block(so_ref[...])

    spec = pl.BlockSpec((block, d), lambda i: (i, 0))
    return pl.pallas_call(
        body,
        grid=(n // block,),
        in_specs=[spec, spec],
        out_specs=spec,
        out_shape=jax.ShapeDtypeStruct((n, d), jnp.float32),
    )(x, s_ones)


def _tc_mid(s, y, s_ones, w, b, block):
    n, d = y.shape

    def body(s_ref, y_ref, so_ref, w_ref, b_ref, h_ref, yn_ref):
        dinv = _dinv_block(so_ref[...])
        agg = (s_ref[...] + y_ref[...]) * dinv
        h = jnp.dot(agg, w_ref[...], preferred_element_type=jnp.float32)
        h = h + b_ref[...]
        h_ref[...] = h
        yn_ref[...] = jnp.maximum(h, 0.0) * dinv

    spec = pl.BlockSpec((block, d), lambda i: (i, 0))
    return pl.pallas_call(
        body,
        grid=(n // block,),
        in_specs=[
            spec, spec, spec,
            pl.BlockSpec((d, d), lambda i: (0, 0)),
            pl.BlockSpec((1, d), lambda i: (0, 0)),
        ],
        out_specs=[spec, spec],
        out_shape=[
            jax.ShapeDtypeStruct((n, d), jnp.float32),
            jax.ShapeDtypeStruct((n, d), jnp.float32),
        ],
    )(s, y, s_ones, w, b.reshape(1, d))


def _tc_final(s, y, s_ones, w2, b2, wp1, bp1, wp2, bp2, block):
    n, d = y.shape

    def body(s_ref, y_ref, so_ref, w2_ref, b2_ref, wp1_ref, bp1_ref,
             wp2_ref, bp2_ref, h2_ref, out_ref):
        dinv = _dinv_block(so_ref[...])
        agg = (s_ref[...] + y_ref[...]) * dinv
        h2 = jnp.dot(agg, w2_ref[...], preferred_element_type=jnp.float32)
        h2 = h2 + b2_ref[...]
        h2_ref[...] = h2
        r = jnp.maximum(h2, 0.0)
        t = jnp.dot(r, wp1_ref[...], preferred_element_type=jnp.float32)
        t = t + bp1_ref[...]
        p = jnp.dot(t, wp2_ref[...], preferred_element_type=jnp.float32)
        p = p + bp2_ref[...]
        m = jnp.max(p, axis=1, keepdims=True)
        lse = jnp.log(jnp.sum(jnp.exp(p - m), axis=1, keepdims=True)) + m
        out_ref[...] = p - lse

    spec = pl.BlockSpec((block, d), lambda i: (i, 0))
    wspec = pl.BlockSpec((d, d), lambda i: (0, 0))
    bspec = pl.BlockSpec((1, d), lambda i: (0, 0))
    return pl.pallas_call(
        body,
        grid=(n // block,),
        in_specs=[spec, spec, spec, wspec, bspec, wspec, bspec, wspec, bspec],
        out_specs=[spec, spec],
        out_shape=[
            jax.ShapeDtypeStruct((n, d), jnp.float32),
            jax.ShapeDtypeStruct((n, d), jnp.float32),
        ],
    )(s, y, s_ones, w2, b2.reshape(1, d), wp1, bp1.reshape(1, d),
      wp2, bp2.reshape(1, d))


@jax.jit
def kernel(x, edge_index, batch, W1, b1, W2, b2, Wp1, bp1, Wp2, bp2):
    n, d = x.shape
    e = edge_index.shape[1]
    nbatch = -(-e // (NS * K * CB))
    chunks = nbatch * CB
    epad = chunks * NS * K - e
    pad_i = jnp.arange(epad, dtype=jnp.int32)
    src3 = jnp.concatenate([edge_index[0], pad_i % 64]).reshape(NS, chunks, K)
    dst3 = jnp.concatenate([edge_index[1], n + (pad_i % 8)]).reshape(NS, chunks, K)
    zeros = jnp.zeros((n + 8, d), jnp.float32)

    agg_ones = _make_aggregate_kernel(n, d, nbatch, False)
    agg = _make_aggregate_kernel(n, d, nbatch, True)

    block = 1000
    s_ones = agg_ones(dst3, zeros)
    y1 = _tc_prep(x, s_ones, block)
    s1 = agg(y1, src3, dst3, zeros)
    h1, y2 = _tc_mid(s1, y1, s_ones, W1, b1, block)
    s2 = agg(y2, src3, dst3, zeros)
    h2, out = _tc_final(s2, y2, s_ones, W2, b2, Wp1, bp1, Wp2, bp2, block)
    return (h1, h2, out)
